# Initial kernel scaffold; baseline (speedup 1.0000x reference)
import jax
import jax.numpy as jnp
from jax.experimental import pallas as pl


def kernel(edge_index, edge_type, emb, W1, W2):
    raise NotImplementedError("write your pallas kernel here")


# speedup vs baseline: 38.6514x; 38.6514x over previous
import functools

import jax
import jax.numpy as jnp
from jax import lax
from jax.experimental import pallas as pl
from jax.experimental.pallas import tpu as pltpu
import jax.experimental.pallas.tpu_sc as plsc

N_NODES = 10000
N_EDGES = 320000
NREL = 16
FEAT = 128
HID = 16
NCLS = 4

NC = 2
NS = 16
LANES = 16
NW = NC * NS
EPW = N_EDGES // NW
CHUNK = 80
NCHUNKS = EPW // CHUNK
VPC = CHUNK // LANES
NSEG = NREL * N_NODES
SEG_PER_TILE = NSEG // NS
CP_TILES = 10
CP_ROWS = N_NODES // CP_TILES

_MESH = plsc.VectorSubcoreMesh(
    core_axis_name="c", subcore_axis_name="s", num_cores=NC, num_subcores=NS)
_SC_PARAMS = pltpu.CompilerParams(use_tc_tiling_on_sc=False)


@functools.partial(
    pl.kernel,
    out_type=jax.ShapeDtypeStruct((NC * NSEG,), jnp.float32),
    mesh=_MESH,
    compiler_params=_SC_PARAMS,
    scratch_types=[
        pltpu.VMEM((NCHUNKS, CHUNK), jnp.int32),
        pltpu.VMEM((NCHUNKS, CHUNK), jnp.int32),
        pltpu.VMEM((NCHUNKS, CHUNK), jnp.float32),
        pltpu.VMEM((SEG_PER_TILE,), jnp.float32),
        pltpu.VMEM_SHARED((NSEG,), jnp.float32),
    ],
)
def _count_kernel(dst_hbm, et_hbm, cnt_hbm, dv2, tv2, ones2, cbuf, cnt_sh):
    c = lax.axis_index("c")
    s = lax.axis_index("s")
    wid = c * NS + s

    def zero_body(i, _):
        cbuf[pl.ds(i * LANES, LANES)] = jnp.zeros((LANES,), jnp.float32)
        return 0

    lax.fori_loop(0, SEG_PER_TILE // LANES, zero_body, 0)
    pltpu.sync_copy(cbuf, cnt_sh.at[pl.ds(s * SEG_PER_TILE, SEG_PER_TILE)])

    pltpu.sync_copy(dst_hbm.at[wid], dv2)
    pltpu.sync_copy(et_hbm.at[wid], tv2)

    one_v = jnp.ones((LANES,), jnp.float32)

    def idx_body(r, _):
        for j in range(VPC):
            d16 = dv2[r, pl.ds(j * LANES, LANES)]
            t16 = tv2[r, pl.ds(j * LANES, LANES)]
            tv2[r, pl.ds(j * LANES, LANES)] = t16 * N_NODES + d16
            ones2[r, pl.ds(j * LANES, LANES)] = one_v
        return 0

    lax.fori_loop(0, NCHUNKS, idx_body, 0)
    plsc.subcore_barrier()

    def body(g, _):
        pltpu.sync_copy(ones2.at[g], cnt_sh.at[tv2.at[g]], add=True)
        return 0

    lax.fori_loop(0, NCHUNKS, body, 0)
    plsc.subcore_barrier()
    pltpu.sync_copy(cnt_sh.at[pl.ds(s * SEG_PER_TILE, SEG_PER_TILE)], cbuf)
    pltpu.sync_copy(
        cbuf, cnt_hbm.at[pl.ds(c * NSEG + s * SEG_PER_TILE, SEG_PER_TILE)])


@functools.partial(
    pl.kernel,
    out_type=jax.ShapeDtypeStruct((NC * N_NODES, LANES), jnp.float32),
    mesh=_MESH,
    compiler_params=_SC_PARAMS,
    scratch_types=[
        pltpu.VMEM((NCHUNKS, CHUNK), jnp.int32),
        pltpu.VMEM((NCHUNKS, CHUNK), jnp.int32),
        pltpu.VMEM((NCHUNKS, CHUNK), jnp.int32),
        pltpu.VMEM((CHUNK, LANES), jnp.float32),
        pltpu.VMEM((CHUNK, LANES), jnp.float32),
        pltpu.VMEM((CHUNK, LANES), jnp.float32),
        pltpu.VMEM((CHUNK, LANES), jnp.float32),
        pltpu.VMEM((CP_ROWS, LANES), jnp.float32),
        pltpu.VMEM_SHARED((N_NODES, LANES), jnp.float32),
        pltpu.SemaphoreType.DMA,
        pltpu.SemaphoreType.DMA,
        pltpu.SemaphoreType.DMA,
        pltpu.SemaphoreType.DMA,
        pltpu.SemaphoreType.DMA,
        pltpu.SemaphoreType.DMA,
    ],
)
def _aggregate_kernel(src_hbm, dst_hbm, et_hbm, h_hbm, inv_hbm, out_hbm,
                      gi2, dv2, si2, rows0, wrow0, rows1, wrow1, obuf, acc_sh,
                      sgr0, sgw0, sgr1, sgw1, ss0, ss1):
    c = lax.axis_index("c")
    s = lax.axis_index("s")
    wid = c * NS + s

    @pl.when(s < CP_TILES)
    def _():
        def zero_body(i, _):
            obuf[i] = jnp.zeros((LANES,), jnp.float32)
            return 0

        lax.fori_loop(0, CP_ROWS, zero_body, 0)
        pltpu.sync_copy(obuf, acc_sh.at[pl.ds(s * CP_ROWS, CP_ROWS)])

    pltpu.sync_copy(src_hbm.at[wid], gi2)
    pltpu.sync_copy(dst_hbm.at[wid], dv2)
    pltpu.sync_copy(et_hbm.at[wid], si2)

    def idx_body(r, _):
        for j in range(VPC):
            s16 = gi2[r, pl.ds(j * LANES, LANES)]
            d16 = dv2[r, pl.ds(j * LANES, LANES)]
            t16 = si2[r, pl.ds(j * LANES, LANES)]
            gi2[r, pl.ds(j * LANES, LANES)] = s16 * NREL + t16
            si2[r, pl.ds(j * LANES, LANES)] = t16 * N_NODES + d16
        return 0

    lax.fori_loop(0, NCHUNKS, idx_body, 0)
    plsc.subcore_barrier()

    def issue_gather(g, rows, wrow, sr, sw):
        pltpu.async_copy(h_hbm.at[gi2.at[g]], rows, sr)
        pltpu.async_copy(inv_hbm.at[si2.at[g]], wrow, sw)

    def wait_gather(g, rows, wrow, sr, sw):
        pltpu.make_async_copy(h_hbm.at[gi2.at[g]], rows, sr).wait()
        pltpu.make_async_copy(inv_hbm.at[si2.at[g]], wrow, sw).wait()

    def scale(rows, wrow):
        for e in range(CHUNK):
            rows[e] = rows[e] * wrow[e]

    issue_gather(0, rows0, wrow0, sgr0, sgw0)

    def body(i, _):
        a = 2 * i
        b = a + 1

        @pl.when(i > 0)
        def _():
            pltpu.make_async_copy(rows1, acc_sh.at[dv2.at[a - 1]], ss1).wait()

        @pl.when(b < NCHUNKS)
        def _():
            issue_gather(b, rows1, wrow1, sgr1, sgw1)

        wait_gather(a, rows0, wrow0, sgr0, sgw0)
        scale(rows0, wrow0)
        pltpu.async_copy(rows0, acc_sh.at[dv2.at[a]], ss0, add=True)

        @pl.when(b < NCHUNKS)
        def _():
            wait_gather(b, rows1, wrow1, sgr1, sgw1)
            scale(rows1, wrow1)
            pltpu.async_copy(rows1, acc_sh.at[dv2.at[b]], ss1, add=True)

        pltpu.make_async_copy(rows0, acc_sh.at[dv2.at[a]], ss0).wait()

        @pl.when(a + 2 < NCHUNKS)
        def _():
            issue_gather(a + 2, rows0, wrow0, sgr0, sgw0)

        return 0

    lax.fori_loop(0, (NCHUNKS + 1) // 2, body, 0)
    plsc.subcore_barrier()

    @pl.when(s < CP_TILES)
    def _():
        pltpu.sync_copy(acc_sh.at[pl.ds(s * CP_ROWS, CP_ROWS)], obuf)
        pltpu.sync_copy(
            obuf, out_hbm.at[pl.ds(c * N_NODES + s * CP_ROWS, CP_ROWS)])


def _mm_body(x_ref, w_ref, o_ref):
    o_ref[...] = jnp.dot(x_ref[...], w_ref[...],
                         preferred_element_type=jnp.float32)


def _matmul(x, w, bm):
    n, k = x.shape
    _, m = w.shape
    return pl.pallas_call(
        _mm_body,
        grid=(n // bm,),
        in_specs=[
            pl.BlockSpec((bm, k), lambda i: (i, 0)),
            pl.BlockSpec((k, m), lambda i: (0, 0)),
        ],
        out_specs=pl.BlockSpec((bm, m), lambda i: (i, 0)),
        out_shape=jax.ShapeDtypeStruct((n, m), jnp.float32),
    )(x, w)


_INV_BM = 6400


def _inv_body(c_ref, o_ref):
    tot = c_ref[0, :] + c_ref[1, :]
    inv = 1.0 / jnp.maximum(tot, 1.0)
    o_ref[...] = jnp.broadcast_to(inv[:, None], (_INV_BM, LANES))


def _inv_bcast(cnt):
    return pl.pallas_call(
        _inv_body,
        grid=(NSEG // _INV_BM,),
        in_specs=[pl.BlockSpec((NC, _INV_BM), lambda i: (0, i))],
        out_specs=pl.BlockSpec((_INV_BM, LANES), lambda i: (i, 0)),
        out_shape=jax.ShapeDtypeStruct((NSEG, LANES), jnp.float32),
    )(cnt)


_L2_BM = 2000


def _relu_mm_body(p_ref, w_ref, o_ref):
    x = jnp.maximum(p_ref[0] + p_ref[1], 0.0)
    o_ref[...] = jnp.dot(x, w_ref[...], preferred_element_type=jnp.float32)


def _relu_matmul(p, w):
    m = w.shape[1]
    return pl.pallas_call(
        _relu_mm_body,
        grid=(N_NODES // _L2_BM,),
        in_specs=[
            pl.BlockSpec((NC, _L2_BM, HID), lambda i: (0, i, 0)),
            pl.BlockSpec((HID, m), lambda i: (0, 0)),
        ],
        out_specs=pl.BlockSpec((_L2_BM, m), lambda i: (i, 0)),
        out_shape=jax.ShapeDtypeStruct((N_NODES, m), jnp.float32),
    )(p, w)


def _lsm_body(p_ref, o_ref):
    x = p_ref[0] + p_ref[1]
    logits = x[:, :NCLS]
    m = jnp.max(logits, axis=1, keepdims=True)
    z = logits - m
    lse = jnp.log(jnp.sum(jnp.exp(z), axis=1, keepdims=True))
    o_ref[...] = z - lse


def _log_softmax(p):
    return pl.pallas_call(
        _lsm_body,
        grid=(N_NODES // _L2_BM,),
        in_specs=[pl.BlockSpec((NC, _L2_BM, LANES), lambda i: (0, i, 0))],
        out_specs=pl.BlockSpec((_L2_BM, NCLS), lambda i: (i, 0)),
        out_shape=jax.ShapeDtypeStruct((N_NODES, NCLS), jnp.float32),
    )(p)


def kernel(edge_index, edge_type, emb, W1, W2):
    src = edge_index[0].reshape(NW, NCHUNKS, CHUNK)
    dst = edge_index[1].reshape(NW, NCHUNKS, CHUNK)
    et = edge_type.reshape(NW, NCHUNKS, CHUNK)

    w1cat = W1.transpose(1, 0, 2).reshape(FEAT, NREL * HID)
    w2pad = jnp.pad(W2, ((0, 0), (0, 0), (0, LANES - NCLS)))
    w2cat = w2pad.transpose(1, 0, 2).reshape(HID, NREL * LANES)

    cnt = _count_kernel(dst, et).reshape(NC, NSEG)
    inv = _inv_bcast(cnt)

    h1 = _matmul(emb, w1cat, 2000).reshape(N_NODES * NREL, HID)
    p1 = _aggregate_kernel(src, dst, et, h1, inv)
    p1 = p1.reshape(NC, N_NODES, LANES)

    h2 = _relu_matmul(p1, w2cat).reshape(N_NODES * NREL, LANES)
    p2 = _aggregate_kernel(src, dst, et, h2, inv)
    p2 = p2.reshape(NC, N_NODES, LANES)

    return _log_softmax(p2)


# speedup vs baseline: 48.4997x; 1.2548x over previous
import functools

import jax
import jax.numpy as jnp
from jax import lax
from jax.experimental import pallas as pl
from jax.experimental.pallas import tpu as pltpu
import jax.experimental.pallas.tpu_sc as plsc

N_NODES = 10000
N_EDGES = 320000
NREL = 16
FEAT = 128
HID = 16
NCLS = 4

NC = 2
NS = 16
LANES = 16
NW = NC * NS
EPW = N_EDGES // NW
CHUNK = 80
NCHUNKS = EPW // CHUNK
VPC = CHUNK // LANES
NSEG = NREL * N_NODES
SEG_PER_TILE = NSEG // NS
CP_TILES = 10
CP_ROWS = N_NODES // CP_TILES

_MESH = plsc.VectorSubcoreMesh(
    core_axis_name="c", subcore_axis_name="s", num_cores=NC, num_subcores=NS)
_SC_PARAMS = pltpu.CompilerParams(use_tc_tiling_on_sc=False)


@functools.partial(
    pl.kernel,
    out_type=jax.ShapeDtypeStruct((NC * NSEG,), jnp.float32),
    mesh=_MESH,
    compiler_params=_SC_PARAMS,
    scratch_types=[
        pltpu.VMEM((NCHUNKS, CHUNK), jnp.int32),
        pltpu.VMEM((NCHUNKS, CHUNK), jnp.int32),
        pltpu.VMEM((NCHUNKS, CHUNK), jnp.float32),
        pltpu.VMEM((SEG_PER_TILE,), jnp.float32),
        pltpu.VMEM_SHARED((NSEG,), jnp.float32),
    ],
)
def _count_kernel(dst_hbm, et_hbm, cnt_hbm, dv2, tv2, ones2, cbuf, cnt_sh):
    c = lax.axis_index("c")
    s = lax.axis_index("s")
    wid = c * NS + s

    def zero_body(i, _):
        cbuf[pl.ds(i * LANES, LANES)] = jnp.zeros((LANES,), jnp.float32)
        return 0

    lax.fori_loop(0, SEG_PER_TILE // LANES, zero_body, 0)
    pltpu.sync_copy(cbuf, cnt_sh.at[pl.ds(s * SEG_PER_TILE, SEG_PER_TILE)])

    pltpu.sync_copy(dst_hbm.at[wid], dv2)
    pltpu.sync_copy(et_hbm.at[wid], tv2)

    one_v = jnp.ones((LANES,), jnp.float32)

    def idx_body(r, _):
        for j in range(VPC):
            d16 = dv2[r, pl.ds(j * LANES, LANES)]
            t16 = tv2[r, pl.ds(j * LANES, LANES)]
            tv2[r, pl.ds(j * LANES, LANES)] = t16 * N_NODES + d16
            ones2[r, pl.ds(j * LANES, LANES)] = one_v
        return 0

    lax.fori_loop(0, NCHUNKS, idx_body, 0)
    plsc.subcore_barrier()

    def body(g, _):
        pltpu.sync_copy(ones2.at[g], cnt_sh.at[tv2.at[g]], add=True)
        return 0

    lax.fori_loop(0, NCHUNKS, body, 0)
    plsc.subcore_barrier()
    pltpu.sync_copy(cnt_sh.at[pl.ds(s * SEG_PER_TILE, SEG_PER_TILE)], cbuf)
    pltpu.sync_copy(
        cbuf, cnt_hbm.at[pl.ds(c * NSEG + s * SEG_PER_TILE, SEG_PER_TILE)])


@functools.partial(
    pl.kernel,
    out_type=jax.ShapeDtypeStruct((NC * N_NODES, LANES), jnp.float32),
    mesh=_MESH,
    compiler_params=_SC_PARAMS,
    scratch_types=[
        pltpu.VMEM((NCHUNKS, CHUNK), jnp.int32),
        pltpu.VMEM((NCHUNKS, CHUNK), jnp.int32),
        pltpu.VMEM((NCHUNKS, CHUNK), jnp.int32),
        pltpu.VMEM((NCHUNKS, CHUNK), jnp.int32),
        pltpu.VMEM((CHUNK, LANES), jnp.float32),
        pltpu.VMEM((CHUNK,), jnp.float32),
        pltpu.VMEM((CHUNK,), jnp.float32),
        pltpu.VMEM((CHUNK, LANES), jnp.float32),
        pltpu.VMEM((CHUNK,), jnp.float32),
        pltpu.VMEM((CHUNK,), jnp.float32),
        pltpu.VMEM((CP_ROWS, LANES), jnp.float32),
        pltpu.VMEM_SHARED((N_NODES, LANES), jnp.float32),
        pltpu.SemaphoreType.DMA,
        pltpu.SemaphoreType.DMA,
        pltpu.SemaphoreType.DMA,
        pltpu.SemaphoreType.DMA,
        pltpu.SemaphoreType.DMA,
        pltpu.SemaphoreType.DMA,
    ],
)
def _aggregate_kernel(src_hbm, dst_hbm, et_hbm, h_hbm, cnt_hbm, out_hbm,
                      gi2, dv2, si2, si2b, rows0, c0v0, c1v0,
                      rows1, c0v1, c1v1, obuf, acc_sh,
                      sgr0, sgw0, sgr1, sgw1, ss0, ss1):
    c = lax.axis_index("c")
    s = lax.axis_index("s")
    wid = c * NS + s

    @pl.when(s < CP_TILES)
    def _():
        def zero_body(i, _):
            obuf[i] = jnp.zeros((LANES,), jnp.float32)
            return 0

        lax.fori_loop(0, CP_ROWS, zero_body, 0)
        pltpu.sync_copy(obuf, acc_sh.at[pl.ds(s * CP_ROWS, CP_ROWS)])

    pltpu.sync_copy(src_hbm.at[wid], gi2)
    pltpu.sync_copy(dst_hbm.at[wid], dv2)
    pltpu.sync_copy(et_hbm.at[wid], si2)

    def idx_body(r, _):
        for j in range(VPC):
            s16 = gi2[r, pl.ds(j * LANES, LANES)]
            d16 = dv2[r, pl.ds(j * LANES, LANES)]
            t16 = si2[r, pl.ds(j * LANES, LANES)]
            seg = t16 * N_NODES + d16
            gi2[r, pl.ds(j * LANES, LANES)] = s16 * NREL + t16
            si2[r, pl.ds(j * LANES, LANES)] = seg
            si2b[r, pl.ds(j * LANES, LANES)] = seg + NSEG
        return 0

    lax.fori_loop(0, NCHUNKS, idx_body, 0)
    plsc.subcore_barrier()

    def issue_gather(g, rows, c0v, c1v, sr, sw):
        pltpu.async_copy(h_hbm.at[gi2.at[g]], rows, sr)
        pltpu.async_copy(cnt_hbm.at[si2.at[g]], c0v, sw)
        pltpu.async_copy(cnt_hbm.at[si2b.at[g]], c1v, sw)

    def wait_gather(g, rows, c0v, c1v, sr, sw):
        pltpu.make_async_copy(h_hbm.at[gi2.at[g]], rows, sr).wait()
        pltpu.make_async_copy(cnt_hbm.at[si2.at[g]], c0v, sw).wait()
        pltpu.make_async_copy(cnt_hbm.at[si2b.at[g]], c1v, sw).wait()

    def scale(rows, c0v, c1v):
        for j in range(VPC):
            a0 = c0v[pl.ds(j * LANES, LANES)]
            a1 = c1v[pl.ds(j * LANES, LANES)]
            wv = 1.0 / jnp.maximum(a0 + a1, 1.0)
            for l in range(LANES):
                e = j * LANES + l
                rows[e] = rows[e] * wv[l]

    issue_gather(0, rows0, c0v0, c1v0, sgr0, sgw0)

    def body(i, _):
        a = 2 * i
        b = a + 1

        @pl.when(i > 0)
        def _():
            pltpu.make_async_copy(rows1, acc_sh.at[dv2.at[a - 1]], ss1).wait()

        @pl.when(b < NCHUNKS)
        def _():
            issue_gather(b, rows1, c0v1, c1v1, sgr1, sgw1)

        wait_gather(a, rows0, c0v0, c1v0, sgr0, sgw0)
        scale(rows0, c0v0, c1v0)
        pltpu.async_copy(rows0, acc_sh.at[dv2.at[a]], ss0, add=True)

        @pl.when(b < NCHUNKS)
        def _():
            wait_gather(b, rows1, c0v1, c1v1, sgr1, sgw1)
            scale(rows1, c0v1, c1v1)
            pltpu.async_copy(rows1, acc_sh.at[dv2.at[b]], ss1, add=True)

        pltpu.make_async_copy(rows0, acc_sh.at[dv2.at[a]], ss0).wait()

        @pl.when(a + 2 < NCHUNKS)
        def _():
            issue_gather(a + 2, rows0, c0v0, c1v0, sgr0, sgw0)

        return 0

    lax.fori_loop(0, (NCHUNKS + 1) // 2, body, 0)
    plsc.subcore_barrier()

    @pl.when(s < CP_TILES)
    def _():
        pltpu.sync_copy(acc_sh.at[pl.ds(s * CP_ROWS, CP_ROWS)], obuf)
        pltpu.sync_copy(
            obuf, out_hbm.at[pl.ds(c * N_NODES + s * CP_ROWS, CP_ROWS)])


def _mm_body(x_ref, w_ref, o_ref):
    o_ref[...] = jnp.dot(x_ref[...], w_ref[...],
                         preferred_element_type=jnp.float32)


def _matmul(x, w, bm):
    n, k = x.shape
    _, m = w.shape
    return pl.pallas_call(
        _mm_body,
        grid=(n // bm,),
        in_specs=[
            pl.BlockSpec((bm, k), lambda i: (i, 0)),
            pl.BlockSpec((k, m), lambda i: (0, 0)),
        ],
        out_specs=pl.BlockSpec((bm, m), lambda i: (i, 0)),
        out_shape=jax.ShapeDtypeStruct((n, m), jnp.float32),
    )(x, w)


_L2_BM = 2000


def _relu_mm_body(p_ref, w_ref, o_ref):
    x = jnp.maximum(p_ref[0] + p_ref[1], 0.0)
    o_ref[...] = jnp.dot(x, w_ref[...], preferred_element_type=jnp.float32)


def _relu_matmul(p, w):
    m = w.shape[1]
    return pl.pallas_call(
        _relu_mm_body,
        grid=(N_NODES // _L2_BM,),
        in_specs=[
            pl.BlockSpec((NC, _L2_BM, HID), lambda i: (0, i, 0)),
            pl.BlockSpec((HID, m), lambda i: (0, 0)),
        ],
        out_specs=pl.BlockSpec((_L2_BM, m), lambda i: (i, 0)),
        out_shape=jax.ShapeDtypeStruct((N_NODES, m), jnp.float32),
    )(p, w)


def _lsm_body(p_ref, o_ref):
    x = p_ref[0] + p_ref[1]
    logits = x[:, :NCLS]
    m = jnp.max(logits, axis=1, keepdims=True)
    z = logits - m
    lse = jnp.log(jnp.sum(jnp.exp(z), axis=1, keepdims=True))
    o_ref[...] = z - lse


def _log_softmax(p):
    return pl.pallas_call(
        _lsm_body,
        grid=(N_NODES // _L2_BM,),
        in_specs=[pl.BlockSpec((NC, _L2_BM, LANES), lambda i: (0, i, 0))],
        out_specs=pl.BlockSpec((_L2_BM, NCLS), lambda i: (i, 0)),
        out_shape=jax.ShapeDtypeStruct((N_NODES, NCLS), jnp.float32),
    )(p)


def kernel(edge_index, edge_type, emb, W1, W2):
    src = edge_index[0].reshape(NW, NCHUNKS, CHUNK)
    dst = edge_index[1].reshape(NW, NCHUNKS, CHUNK)
    et = edge_type.reshape(NW, NCHUNKS, CHUNK)

    w1cat = W1.transpose(1, 0, 2).reshape(FEAT, NREL * HID)
    w2pad = jnp.pad(W2, ((0, 0), (0, 0), (0, LANES - NCLS)))
    w2cat = w2pad.transpose(1, 0, 2).reshape(HID, NREL * LANES)

    cnt = _count_kernel(dst, et)

    h1 = _matmul(emb, w1cat, 2000).reshape(N_NODES * NREL, HID)
    p1 = _aggregate_kernel(src, dst, et, h1, cnt)
    p1 = p1.reshape(NC, N_NODES, LANES)

    h2 = _relu_matmul(p1, w2cat).reshape(N_NODES * NREL, LANES)
    p2 = _aggregate_kernel(src, dst, et, h2, cnt)
    p2 = p2.reshape(NC, N_NODES, LANES)

    return _log_softmax(p2)


# speedup vs baseline: 61.2265x; 1.2624x over previous
import functools

import jax
import jax.numpy as jnp
from jax import lax
from jax.experimental import pallas as pl
from jax.experimental.pallas import tpu as pltpu
import jax.experimental.pallas.tpu_sc as plsc

N_NODES = 10000
N_EDGES = 320000
NREL = 16
FEAT = 128
HID = 16
NCLS = 4

NC = 2
NS = 16
LANES = 16
NW = NC * NS
EPW = N_EDGES // NW
CHUNK = 400
NCHUNKS = EPW // CHUNK
VPC = CHUNK // LANES
NSEG = NREL * N_NODES
SEG_PER_TILE = NSEG // NS
CP_TILES = 10
CP_ROWS = N_NODES // CP_TILES

_MESH = plsc.VectorSubcoreMesh(
    core_axis_name="c", subcore_axis_name="s", num_cores=NC, num_subcores=NS)
_SC_PARAMS = pltpu.CompilerParams(use_tc_tiling_on_sc=False)


@functools.partial(
    pl.kernel,
    out_type=jax.ShapeDtypeStruct((NC * NSEG,), jnp.float32),
    mesh=_MESH,
    compiler_params=_SC_PARAMS,
    scratch_types=[
        pltpu.VMEM((NCHUNKS, CHUNK), jnp.int32),
        pltpu.VMEM((NCHUNKS, CHUNK), jnp.int32),
        pltpu.VMEM((NCHUNKS, CHUNK), jnp.float32),
        pltpu.VMEM((SEG_PER_TILE,), jnp.float32),
        pltpu.VMEM_SHARED((NSEG,), jnp.float32),
    ],
)
def _count_kernel(dst_hbm, et_hbm, cnt_hbm, dv2, tv2, ones2, cbuf, cnt_sh):
    c = lax.axis_index("c")
    s = lax.axis_index("s")
    wid = c * NS + s

    def zero_body(i, _):
        cbuf[pl.ds(i * LANES, LANES)] = jnp.zeros((LANES,), jnp.float32)
        return 0

    lax.fori_loop(0, SEG_PER_TILE // LANES, zero_body, 0)
    pltpu.sync_copy(cbuf, cnt_sh.at[pl.ds(s * SEG_PER_TILE, SEG_PER_TILE)])

    pltpu.sync_copy(dst_hbm.at[wid], dv2)
    pltpu.sync_copy(et_hbm.at[wid], tv2)

    one_v = jnp.ones((LANES,), jnp.float32)

    def idx_body(r, _):
        for j in range(VPC):
            d16 = dv2[r, pl.ds(j * LANES, LANES)]
            t16 = tv2[r, pl.ds(j * LANES, LANES)]
            tv2[r, pl.ds(j * LANES, LANES)] = t16 * N_NODES + d16
            ones2[r, pl.ds(j * LANES, LANES)] = one_v
        return 0

    lax.fori_loop(0, NCHUNKS, idx_body, 0)
    plsc.subcore_barrier()

    def body(g, _):
        pltpu.sync_copy(ones2.at[g], cnt_sh.at[tv2.at[g]], add=True)
        return 0

    lax.fori_loop(0, NCHUNKS, body, 0)
    plsc.subcore_barrier()
    pltpu.sync_copy(cnt_sh.at[pl.ds(s * SEG_PER_TILE, SEG_PER_TILE)], cbuf)
    pltpu.sync_copy(
        cbuf, cnt_hbm.at[pl.ds(c * NSEG + s * SEG_PER_TILE, SEG_PER_TILE)])


@functools.partial(
    pl.kernel,
    out_type=jax.ShapeDtypeStruct((NC * N_NODES, LANES), jnp.float32),
    mesh=_MESH,
    compiler_params=_SC_PARAMS,
    scratch_types=[
        pltpu.VMEM((NCHUNKS, CHUNK), jnp.int32),
        pltpu.VMEM((NCHUNKS, CHUNK), jnp.int32),
        pltpu.VMEM((NCHUNKS, CHUNK), jnp.int32),
        pltpu.VMEM((NCHUNKS, CHUNK), jnp.int32),
        pltpu.VMEM((CHUNK, LANES), jnp.float32),
        pltpu.VMEM((CHUNK,), jnp.float32),
        pltpu.VMEM((CHUNK,), jnp.float32),
        pltpu.VMEM((CHUNK, LANES), jnp.float32),
        pltpu.VMEM((CHUNK,), jnp.float32),
        pltpu.VMEM((CHUNK,), jnp.float32),
        pltpu.VMEM((CP_ROWS, LANES), jnp.float32),
        pltpu.VMEM_SHARED((N_NODES, LANES), jnp.float32),
        pltpu.SemaphoreType.DMA,
        pltpu.SemaphoreType.DMA,
        pltpu.SemaphoreType.DMA,
        pltpu.SemaphoreType.DMA,
        pltpu.SemaphoreType.DMA,
        pltpu.SemaphoreType.DMA,
    ],
)
def _aggregate_kernel(src_hbm, dst_hbm, et_hbm, h_hbm, cnt_hbm, out_hbm,
                      gi2, dv2, si2, si2b, rows0, c0v0, c1v0,
                      rows1, c0v1, c1v1, obuf, acc_sh,
                      sgr0, sgw0, sgr1, sgw1, ss0, ss1):
    c = lax.axis_index("c")
    s = lax.axis_index("s")
    wid = c * NS + s

    @pl.when(s < CP_TILES)
    def _():
        def zero_body(i, _):
            obuf[i] = jnp.zeros((LANES,), jnp.float32)
            return 0

        lax.fori_loop(0, CP_ROWS, zero_body, 0)
        pltpu.sync_copy(obuf, acc_sh.at[pl.ds(s * CP_ROWS, CP_ROWS)])

    pltpu.sync_copy(src_hbm.at[wid], gi2)
    pltpu.sync_copy(dst_hbm.at[wid], dv2)
    pltpu.sync_copy(et_hbm.at[wid], si2)

    def idx_body(r, _):
        for j in range(VPC):
            s16 = gi2[r, pl.ds(j * LANES, LANES)]
            d16 = dv2[r, pl.ds(j * LANES, LANES)]
            t16 = si2[r, pl.ds(j * LANES, LANES)]
            seg = t16 * N_NODES + d16
            gi2[r, pl.ds(j * LANES, LANES)] = s16 * NREL + t16
            si2[r, pl.ds(j * LANES, LANES)] = seg
            si2b[r, pl.ds(j * LANES, LANES)] = seg + NSEG
        return 0

    lax.fori_loop(0, NCHUNKS, idx_body, 0)
    plsc.subcore_barrier()

    def issue_gather(g, rows, c0v, c1v, sr, sw):
        pltpu.async_copy(h_hbm.at[gi2.at[g]], rows, sr)
        pltpu.async_copy(cnt_hbm.at[si2.at[g]], c0v, sw)
        pltpu.async_copy(cnt_hbm.at[si2b.at[g]], c1v, sw)

    def wait_gather(g, rows, c0v, c1v, sr, sw):
        pltpu.make_async_copy(h_hbm.at[gi2.at[g]], rows, sr).wait()
        pltpu.make_async_copy(cnt_hbm.at[si2.at[g]], c0v, sw).wait()
        pltpu.make_async_copy(cnt_hbm.at[si2b.at[g]], c1v, sw).wait()

    def scale(rows, c0v, c1v):
        for j in range(VPC):
            a0 = c0v[pl.ds(j * LANES, LANES)]
            a1 = c1v[pl.ds(j * LANES, LANES)]
            wv = 1.0 / jnp.maximum(a0 + a1, 1.0)
            for l in range(LANES):
                e = j * LANES + l
                rows[e] = rows[e] * wv[l]

    issue_gather(0, rows0, c0v0, c1v0, sgr0, sgw0)

    def body(i, _):
        a = 2 * i
        b = a + 1

        @pl.when(i > 0)
        def _():
            pltpu.make_async_copy(rows1, acc_sh.at[dv2.at[a - 1]], ss1).wait()

        @pl.when(b < NCHUNKS)
        def _():
            issue_gather(b, rows1, c0v1, c1v1, sgr1, sgw1)

        wait_gather(a, rows0, c0v0, c1v0, sgr0, sgw0)
        scale(rows0, c0v0, c1v0)
        pltpu.async_copy(rows0, acc_sh.at[dv2.at[a]], ss0, add=True)

        @pl.when(b < NCHUNKS)
        def _():
            wait_gather(b, rows1, c0v1, c1v1, sgr1, sgw1)
            scale(rows1, c0v1, c1v1)
            pltpu.async_copy(rows1, acc_sh.at[dv2.at[b]], ss1, add=True)

        pltpu.make_async_copy(rows0, acc_sh.at[dv2.at[a]], ss0).wait()

        @pl.when(a + 2 < NCHUNKS)
        def _():
            issue_gather(a + 2, rows0, c0v0, c1v0, sgr0, sgw0)

        return 0

    lax.fori_loop(0, (NCHUNKS + 1) // 2, body, 0)
    plsc.subcore_barrier()

    @pl.when(s < CP_TILES)
    def _():
        pltpu.sync_copy(acc_sh.at[pl.ds(s * CP_ROWS, CP_ROWS)], obuf)
        pltpu.sync_copy(
            obuf, out_hbm.at[pl.ds(c * N_NODES + s * CP_ROWS, CP_ROWS)])


def _mm_body(x_ref, w_ref, o_ref):
    o_ref[...] = jnp.dot(x_ref[...], w_ref[...],
                         preferred_element_type=jnp.float32)


def _matmul(x, w, bm):
    n, k = x.shape
    _, m = w.shape
    return pl.pallas_call(
        _mm_body,
        grid=(n // bm,),
        in_specs=[
            pl.BlockSpec((bm, k), lambda i: (i, 0)),
            pl.BlockSpec((k, m), lambda i: (0, 0)),
        ],
        out_specs=pl.BlockSpec((bm, m), lambda i: (i, 0)),
        out_shape=jax.ShapeDtypeStruct((n, m), jnp.float32),
    )(x, w)


_L2_BM = 2000


def _relu_mm_body(p_ref, w_ref, o_ref):
    x = jnp.maximum(p_ref[0] + p_ref[1], 0.0)
    o_ref[...] = jnp.dot(x, w_ref[...], preferred_element_type=jnp.float32)


def _relu_matmul(p, w):
    m = w.shape[1]
    return pl.pallas_call(
        _relu_mm_body,
        grid=(N_NODES // _L2_BM,),
        in_specs=[
            pl.BlockSpec((NC, _L2_BM, HID), lambda i: (0, i, 0)),
            pl.BlockSpec((HID, m), lambda i: (0, 0)),
        ],
        out_specs=pl.BlockSpec((_L2_BM, m), lambda i: (i, 0)),
        out_shape=jax.ShapeDtypeStruct((N_NODES, m), jnp.float32),
    )(p, w)


def _lsm_body(p_ref, o_ref):
    x = p_ref[0] + p_ref[1]
    logits = x[:, :NCLS]
    m = jnp.max(logits, axis=1, keepdims=True)
    z = logits - m
    lse = jnp.log(jnp.sum(jnp.exp(z), axis=1, keepdims=True))
    o_ref[...] = z - lse


def _log_softmax(p):
    return pl.pallas_call(
        _lsm_body,
        grid=(N_NODES // _L2_BM,),
        in_specs=[pl.BlockSpec((NC, _L2_BM, LANES), lambda i: (0, i, 0))],
        out_specs=pl.BlockSpec((_L2_BM, NCLS), lambda i: (i, 0)),
        out_shape=jax.ShapeDtypeStruct((N_NODES, NCLS), jnp.float32),
    )(p)


def kernel(edge_index, edge_type, emb, W1, W2):
    src = edge_index[0].reshape(NW, NCHUNKS, CHUNK)
    dst = edge_index[1].reshape(NW, NCHUNKS, CHUNK)
    et = edge_type.reshape(NW, NCHUNKS, CHUNK)

    w1cat = W1.transpose(1, 0, 2).reshape(FEAT, NREL * HID)
    w2pad = jnp.pad(W2, ((0, 0), (0, 0), (0, LANES - NCLS)))
    w2cat = w2pad.transpose(1, 0, 2).reshape(HID, NREL * LANES)

    cnt = _count_kernel(dst, et)

    h1 = _matmul(emb, w1cat, 2000).reshape(N_NODES * NREL, HID)
    p1 = _aggregate_kernel(src, dst, et, h1, cnt)
    p1 = p1.reshape(NC, N_NODES, LANES)

    h2 = _relu_matmul(p1, w2cat).reshape(N_NODES * NREL, LANES)
    p2 = _aggregate_kernel(src, dst, et, h2, cnt)
    p2 = p2.reshape(NC, N_NODES, LANES)

    return _log_softmax(p2)


# speedup vs baseline: 65.7566x; 1.0740x over previous
import functools

import jax
import jax.numpy as jnp
from jax import lax
from jax.experimental import pallas as pl
from jax.experimental.pallas import tpu as pltpu
import jax.experimental.pallas.tpu_sc as plsc

N_NODES = 10000
N_EDGES = 320000
NREL = 16
FEAT = 128
HID = 16
NCLS = 4

NC = 2
NS = 16
LANES = 16
NW = NC * NS
EPW = N_EDGES // NW
CHUNK = 400
NCHUNKS = EPW // CHUNK
VPC = CHUNK // LANES
NSEG = NREL * N_NODES
SEG_PER_TILE = NSEG // NS
CP_TILES = 10
CP_ROWS = N_NODES // CP_TILES

_MESH = plsc.VectorSubcoreMesh(
    core_axis_name="c", subcore_axis_name="s", num_cores=NC, num_subcores=NS)
_SC_PARAMS = pltpu.CompilerParams(use_tc_tiling_on_sc=False)


@functools.partial(
    pl.kernel,
    out_type=jax.ShapeDtypeStruct((NC * NSEG,), jnp.float32),
    mesh=_MESH,
    compiler_params=_SC_PARAMS,
    scratch_types=[
        pltpu.VMEM((NCHUNKS, CHUNK), jnp.int32),
        pltpu.VMEM((NCHUNKS, CHUNK), jnp.int32),
        pltpu.VMEM((NCHUNKS, CHUNK), jnp.float32),
        pltpu.VMEM((SEG_PER_TILE,), jnp.float32),
        pltpu.VMEM_SHARED((NSEG,), jnp.float32),
    ],
)
def _count_kernel(dst_hbm, et_hbm, cnt_hbm, dv2, tv2, ones2, cbuf, cnt_sh):
    c = lax.axis_index("c")
    s = lax.axis_index("s")
    wid = c * NS + s

    def zero_body(i, _):
        cbuf[pl.ds(i * LANES, LANES)] = jnp.zeros((LANES,), jnp.float32)
        return 0

    lax.fori_loop(0, SEG_PER_TILE // LANES, zero_body, 0)
    pltpu.sync_copy(cbuf, cnt_sh.at[pl.ds(s * SEG_PER_TILE, SEG_PER_TILE)])

    pltpu.sync_copy(dst_hbm.at[wid], dv2)
    pltpu.sync_copy(et_hbm.at[wid], tv2)

    one_v = jnp.ones((LANES,), jnp.float32)

    def idx_body(r, _):
        for j in range(VPC):
            d16 = dv2[r, pl.ds(j * LANES, LANES)]
            t16 = tv2[r, pl.ds(j * LANES, LANES)]
            tv2[r, pl.ds(j * LANES, LANES)] = t16 * N_NODES + d16
            ones2[r, pl.ds(j * LANES, LANES)] = one_v
        return 0

    lax.fori_loop(0, NCHUNKS, idx_body, 0)
    plsc.subcore_barrier()

    def body(g, _):
        pltpu.sync_copy(ones2.at[g], cnt_sh.at[tv2.at[g]], add=True)
        return 0

    lax.fori_loop(0, NCHUNKS, body, 0)
    plsc.subcore_barrier()
    pltpu.sync_copy(cnt_sh.at[pl.ds(s * SEG_PER_TILE, SEG_PER_TILE)], cbuf)
    pltpu.sync_copy(
        cbuf, cnt_hbm.at[pl.ds(c * NSEG + s * SEG_PER_TILE, SEG_PER_TILE)])


@functools.partial(
    pl.kernel,
    out_type=jax.ShapeDtypeStruct((NC * N_NODES, LANES), jnp.float32),
    mesh=_MESH,
    compiler_params=_SC_PARAMS,
    scratch_types=[
        pltpu.VMEM((NCHUNKS, CHUNK), jnp.int32),
        pltpu.VMEM((NCHUNKS, CHUNK), jnp.int32),
        pltpu.VMEM((NCHUNKS, CHUNK), jnp.int32),
        pltpu.VMEM((CHUNK, LANES), jnp.float32),
        pltpu.VMEM((CHUNK,), jnp.float32),
        pltpu.VMEM((CHUNK, LANES), jnp.float32),
        pltpu.VMEM((CHUNK,), jnp.float32),
        pltpu.VMEM((SEG_PER_TILE,), jnp.float32),
        pltpu.VMEM((SEG_PER_TILE,), jnp.float32),
        pltpu.VMEM((CP_ROWS, LANES), jnp.float32),
        pltpu.VMEM_SHARED((N_NODES, LANES), jnp.float32),
        pltpu.VMEM_SHARED((NSEG,), jnp.float32),
        pltpu.SemaphoreType.DMA,
        pltpu.SemaphoreType.DMA,
        pltpu.SemaphoreType.DMA,
        pltpu.SemaphoreType.DMA,
        pltpu.SemaphoreType.DMA,
        pltpu.SemaphoreType.DMA,
    ],
)
def _aggregate_kernel(src_hbm, dst_hbm, et_hbm, h_hbm, cnt_hbm, out_hbm,
                      gi2, dv2, si2, rows0, wv0, rows1, wv1,
                      c0buf, c1buf, obuf, acc_sh, w_sh,
                      sgr0, sgw0, sgr1, sgw1, ss0, ss1):
    c = lax.axis_index("c")
    s = lax.axis_index("s")
    wid = c * NS + s

    @pl.when(s < CP_TILES)
    def _():
        def zero_body(i, _):
            obuf[i] = jnp.zeros((LANES,), jnp.float32)
            return 0

        lax.fori_loop(0, CP_ROWS, zero_body, 0)
        pltpu.sync_copy(obuf, acc_sh.at[pl.ds(s * CP_ROWS, CP_ROWS)])

    pltpu.sync_copy(cnt_hbm.at[pl.ds(s * SEG_PER_TILE, SEG_PER_TILE)], c0buf)
    pltpu.sync_copy(
        cnt_hbm.at[pl.ds(NSEG + s * SEG_PER_TILE, SEG_PER_TILE)], c1buf)

    def w_body(i, _):
        a0 = c0buf[pl.ds(i * LANES, LANES)]
        a1 = c1buf[pl.ds(i * LANES, LANES)]
        c0buf[pl.ds(i * LANES, LANES)] = 1.0 / jnp.maximum(a0 + a1, 1.0)
        return 0

    lax.fori_loop(0, SEG_PER_TILE // LANES, w_body, 0)
    pltpu.sync_copy(c0buf, w_sh.at[pl.ds(s * SEG_PER_TILE, SEG_PER_TILE)])

    pltpu.sync_copy(src_hbm.at[wid], gi2)
    pltpu.sync_copy(dst_hbm.at[wid], dv2)
    pltpu.sync_copy(et_hbm.at[wid], si2)

    def idx_body(r, _):
        for j in range(VPC):
            s16 = gi2[r, pl.ds(j * LANES, LANES)]
            d16 = dv2[r, pl.ds(j * LANES, LANES)]
            t16 = si2[r, pl.ds(j * LANES, LANES)]
            gi2[r, pl.ds(j * LANES, LANES)] = s16 * NREL + t16
            si2[r, pl.ds(j * LANES, LANES)] = t16 * N_NODES + d16
        return 0

    lax.fori_loop(0, NCHUNKS, idx_body, 0)
    plsc.subcore_barrier()

    def issue_gather(g, rows, wv, sr, sw):
        pltpu.async_copy(h_hbm.at[gi2.at[g]], rows, sr)
        pltpu.async_copy(w_sh.at[si2.at[g]], wv, sw)

    def wait_gather(g, rows, wv, sr, sw):
        pltpu.make_async_copy(h_hbm.at[gi2.at[g]], rows, sr).wait()
        pltpu.make_async_copy(w_sh.at[si2.at[g]], wv, sw).wait()

    def scale(rows, wvr):
        for j in range(VPC):
            wv = wvr[pl.ds(j * LANES, LANES)]
            for l in range(LANES):
                e = j * LANES + l
                rows[e] = rows[e] * wv[l]

    issue_gather(0, rows0, wv0, sgr0, sgw0)

    def body(i, _):
        a = 2 * i
        b = a + 1

        @pl.when(i > 0)
        def _():
            pltpu.make_async_copy(rows1, acc_sh.at[dv2.at[a - 1]], ss1).wait()

        @pl.when(b < NCHUNKS)
        def _():
            issue_gather(b, rows1, wv1, sgr1, sgw1)

        wait_gather(a, rows0, wv0, sgr0, sgw0)
        scale(rows0, wv0)
        pltpu.async_copy(rows0, acc_sh.at[dv2.at[a]], ss0, add=True)

        @pl.when(b < NCHUNKS)
        def _():
            wait_gather(b, rows1, wv1, sgr1, sgw1)
            scale(rows1, wv1)
            pltpu.async_copy(rows1, acc_sh.at[dv2.at[b]], ss1, add=True)

        pltpu.make_async_copy(rows0, acc_sh.at[dv2.at[a]], ss0).wait()

        @pl.when(a + 2 < NCHUNKS)
        def _():
            issue_gather(a + 2, rows0, wv0, sgr0, sgw0)

        return 0

    lax.fori_loop(0, (NCHUNKS + 1) // 2, body, 0)
    plsc.subcore_barrier()

    @pl.when(s < CP_TILES)
    def _():
        pltpu.sync_copy(acc_sh.at[pl.ds(s * CP_ROWS, CP_ROWS)], obuf)
        pltpu.sync_copy(
            obuf, out_hbm.at[pl.ds(c * N_NODES + s * CP_ROWS, CP_ROWS)])


def _mm_body(x_ref, w_ref, o_ref):
    o_ref[...] = jnp.dot(x_ref[...], w_ref[...],
                         preferred_element_type=jnp.float32)


def _matmul(x, w, bm):
    n, k = x.shape
    _, m = w.shape
    return pl.pallas_call(
        _mm_body,
        grid=(n // bm,),
        in_specs=[
            pl.BlockSpec((bm, k), lambda i: (i, 0)),
            pl.BlockSpec((k, m), lambda i: (0, 0)),
        ],
        out_specs=pl.BlockSpec((bm, m), lambda i: (i, 0)),
        out_shape=jax.ShapeDtypeStruct((n, m), jnp.float32),
    )(x, w)


_L2_BM = 2000


def _relu_mm_body(p_ref, w_ref, o_ref):
    x = jnp.maximum(p_ref[0] + p_ref[1], 0.0)
    o_ref[...] = jnp.dot(x, w_ref[...], preferred_element_type=jnp.float32)


def _relu_matmul(p, w):
    m = w.shape[1]
    return pl.pallas_call(
        _relu_mm_body,
        grid=(N_NODES // _L2_BM,),
        in_specs=[
            pl.BlockSpec((NC, _L2_BM, HID), lambda i: (0, i, 0)),
            pl.BlockSpec((HID, m), lambda i: (0, 0)),
        ],
        out_specs=pl.BlockSpec((_L2_BM, m), lambda i: (i, 0)),
        out_shape=jax.ShapeDtypeStruct((N_NODES, m), jnp.float32),
    )(p, w)


def _lsm_body(p_ref, o_ref):
    x = p_ref[0] + p_ref[1]
    logits = x[:, :NCLS]
    m = jnp.max(logits, axis=1, keepdims=True)
    z = logits - m
    lse = jnp.log(jnp.sum(jnp.exp(z), axis=1, keepdims=True))
    o_ref[...] = z - lse


def _log_softmax(p):
    return pl.pallas_call(
        _lsm_body,
        grid=(N_NODES // _L2_BM,),
        in_specs=[pl.BlockSpec((NC, _L2_BM, LANES), lambda i: (0, i, 0))],
        out_specs=pl.BlockSpec((_L2_BM, NCLS), lambda i: (i, 0)),
        out_shape=jax.ShapeDtypeStruct((N_NODES, NCLS), jnp.float32),
    )(p)


def kernel(edge_index, edge_type, emb, W1, W2):
    src = edge_index[0].reshape(NW, NCHUNKS, CHUNK)
    dst = edge_index[1].reshape(NW, NCHUNKS, CHUNK)
    et = edge_type.reshape(NW, NCHUNKS, CHUNK)

    w1cat = W1.transpose(1, 0, 2).reshape(FEAT, NREL * HID)
    w2pad = jnp.pad(W2, ((0, 0), (0, 0), (0, LANES - NCLS)))
    w2cat = w2pad.transpose(1, 0, 2).reshape(HID, NREL * LANES)

    cnt = _count_kernel(dst, et)

    h1 = _matmul(emb, w1cat, 2000).reshape(N_NODES * NREL, HID)
    p1 = _aggregate_kernel(src, dst, et, h1, cnt)
    p1 = p1.reshape(NC, N_NODES, LANES)

    h2 = _relu_matmul(p1, w2cat).reshape(N_NODES * NREL, LANES)
    p2 = _aggregate_kernel(src, dst, et, h2, cnt)
    p2 = p2.reshape(NC, N_NODES, LANES)

    return _log_softmax(p2)
